# baseline (speedup 1.0000x reference)
import functools

import jax
import jax.numpy as jnp
from jax.experimental import pallas as pl
from jax.experimental.pallas import tpu as pltpu

_R = 1024
_E = 256
_EPS = 1e-10
_COMMIT = 0.25


def _vq_tile(x_ref, w_ref, wt_ref,
             qst_ref, enc_ref, idx_ref, hist_ref, sse_ref):
    x = x_ref[...]
    w = w_ref[...]
    xw = jnp.dot(x.astype(jnp.bfloat16), w.astype(jnp.bfloat16),
                 preferred_element_type=jnp.float32)
    w2 = jnp.sum(w * w, axis=0, keepdims=True)
    d = w2 - 2.0 * xw

    dmin = jnp.min(d, axis=1, keepdims=True)
    lane = jax.lax.broadcasted_iota(jnp.int32, (_R, _E), 1).astype(jnp.float32)
    idx = jnp.min(jnp.where(d == dmin, lane, _E), axis=1, keepdims=True)
    enc = (lane == idx).astype(jnp.float32)

    enc_b = enc.astype(jnp.bfloat16)
    q = jnp.dot(enc_b, wt_ref[...].astype(jnp.bfloat16),
                preferred_element_type=jnp.float32)
    qst_ref[...] = q
    enc_ref[...] = enc
    idx_ref[...] = idx.astype(jnp.int32)

    ones_b = jnp.ones((1, _R), jnp.bfloat16)
    hist = jnp.dot(ones_b, enc_b, preferred_element_type=jnp.float32)
    hist_ref[...] = hist.reshape(1, 1, _E)
    sse = jnp.sum(x * x) + jnp.sum(dmin)
    sse_ref[...] = jnp.broadcast_to(sse, (1, 1, 128))


def _vq_finalize(hist_ref, sse_ref, loss_ref, perp_ref, *, n_rows):
    sse = jnp.sum(sse_ref[:, :, 0])
    mse = sse / (n_rows * _E)
    loss_ref[...] = jnp.broadcast_to(mse + _COMMIT * mse, (1, 1))
    p = jnp.sum(hist_ref[:, 0, :], axis=0, keepdims=True) / n_rows
    ent = -jnp.sum(p * jnp.log(p + _EPS), keepdims=True)
    perp_ref[...] = jnp.exp(ent).reshape(1, 1)


def kernel(x, w, is_training):
    lead_shape = x.shape[:-1]
    xf = x.reshape(-1, _E)
    n = xf.shape[0]
    grid = n // _R

    qst, enc, idx, hist_p, sse_p = pl.pallas_call(
        _vq_tile,
        grid=(grid,),
        in_specs=[
            pl.BlockSpec((_R, _E), lambda t: (t, 0)),
            pl.BlockSpec((_E, _E), lambda t: (0, 0)),
            pl.BlockSpec((_E, _E), lambda t: (0, 0)),
        ],
        out_specs=[
            pl.BlockSpec((_R, _E), lambda t: (t, 0)),
            pl.BlockSpec((_R, _E), lambda t: (t, 0)),
            pl.BlockSpec((_R, 1), lambda t: (t, 0)),
            pl.BlockSpec((1, 1, _E), lambda t: (t, 0, 0)),
            pl.BlockSpec((1, 1, 128), lambda t: (t, 0, 0)),
        ],
        out_shape=[
            jax.ShapeDtypeStruct((n, _E), jnp.float32),
            jax.ShapeDtypeStruct((n, _E), jnp.float32),
            jax.ShapeDtypeStruct((n, 1), jnp.int32),
            jax.ShapeDtypeStruct((grid, 1, _E), jnp.float32),
            jax.ShapeDtypeStruct((grid, 1, 128), jnp.float32),
        ],
        compiler_params=pltpu.CompilerParams(
            dimension_semantics=("parallel",)),
    )(xf, w, w.T)

    loss, perp = pl.pallas_call(
        functools.partial(_vq_finalize, n_rows=n),
        out_shape=[
            jax.ShapeDtypeStruct((1, 1), jnp.float32),
            jax.ShapeDtypeStruct((1, 1), jnp.float32),
        ],
    )(hist_p, sse_p)

    return (qst.reshape(x.shape), loss[0, 0], perp[0, 0], enc,
            idx.reshape(lead_shape))


# speedup vs baseline: 1.2801x; 1.2801x over previous
import functools

import jax
import jax.numpy as jnp
from jax.experimental import pallas as pl
from jax.experimental.pallas import tpu as pltpu

_R = 4096
_E = 256
_EPS = 1e-10
_COMMIT = 0.25


def _vq_tile(x_ref, w_ref, wt_ref,
             qst_ref, enc_ref, idx_ref, hist_ref, sse_ref):
    x = x_ref[...]
    w = w_ref[...]
    m2w_b = (-2.0 * w).astype(jnp.bfloat16)
    xw_m2 = jnp.dot(x.astype(jnp.bfloat16), m2w_b,
                    preferred_element_type=jnp.float32)
    w2 = jnp.sum(w * w, axis=0, keepdims=True)
    d = w2 + xw_m2

    dmin = jnp.min(d, axis=1, keepdims=True)
    lane = jax.lax.broadcasted_iota(jnp.int32, (_R, _E), 1).astype(jnp.float32)
    m = d == dmin
    enc = jnp.where(m, 1.0, 0.0)
    idx = jnp.min(jnp.where(m, lane, _E), axis=1, keepdims=True)

    enc_b = enc.astype(jnp.bfloat16)
    q = jnp.dot(enc_b, wt_ref[...].astype(jnp.bfloat16),
                preferred_element_type=jnp.float32)
    qst_ref[...] = q
    enc_ref[...] = enc
    idx_ref[...] = idx.astype(jnp.int32)

    ones_b = jnp.ones((1, _R), jnp.bfloat16)
    hist = jnp.dot(ones_b, enc_b, preferred_element_type=jnp.float32)
    hist_ref[...] = hist.reshape(1, 1, _E)
    sse = jnp.sum(x * x) + jnp.sum(dmin)
    sse_ref[...] = jnp.broadcast_to(sse, (1, 1, 128))


def _vq_finalize(hist_ref, sse_ref, loss_ref, perp_ref, *, n_rows):
    sse = jnp.sum(sse_ref[:, :, 0])
    mse = sse / (n_rows * _E)
    loss_ref[...] = jnp.broadcast_to(mse + _COMMIT * mse, (1, 1))
    p = jnp.sum(hist_ref[:, 0, :], axis=0, keepdims=True) / n_rows
    ent = -jnp.sum(p * jnp.log(p + _EPS), keepdims=True)
    perp_ref[...] = jnp.exp(ent).reshape(1, 1)


def kernel(x, w, is_training):
    lead_shape = x.shape[:-1]
    xf = x.reshape(-1, _E)
    n = xf.shape[0]
    grid = n // _R

    qst, enc, idx, hist_p, sse_p = pl.pallas_call(
        _vq_tile,
        grid=(grid,),
        in_specs=[
            pl.BlockSpec((_R, _E), lambda t: (t, 0)),
            pl.BlockSpec((_E, _E), lambda t: (0, 0)),
            pl.BlockSpec((_E, _E), lambda t: (0, 0)),
        ],
        out_specs=[
            pl.BlockSpec((_R, _E), lambda t: (t, 0)),
            pl.BlockSpec((_R, _E), lambda t: (t, 0)),
            pl.BlockSpec((_R, 1), lambda t: (t, 0)),
            pl.BlockSpec((1, 1, _E), lambda t: (t, 0, 0)),
            pl.BlockSpec((1, 1, 128), lambda t: (t, 0, 0)),
        ],
        out_shape=[
            jax.ShapeDtypeStruct((n, _E), jnp.float32),
            jax.ShapeDtypeStruct((n, _E), jnp.float32),
            jax.ShapeDtypeStruct((n, 1), jnp.int32),
            jax.ShapeDtypeStruct((grid, 1, _E), jnp.float32),
            jax.ShapeDtypeStruct((grid, 1, 128), jnp.float32),
        ],
    )(xf, w, w.T)

    loss, perp = pl.pallas_call(
        functools.partial(_vq_finalize, n_rows=n),
        out_shape=[
            jax.ShapeDtypeStruct((1, 1), jnp.float32),
            jax.ShapeDtypeStruct((1, 1), jnp.float32),
        ],
    )(hist_p, sse_p)

    return (qst.reshape(x.shape), loss[0, 0], perp[0, 0], enc,
            idx.reshape(lead_shape))
